# baseline (speedup 1.0000x reference)
import functools

import jax
import jax.numpy as jnp
from jax.experimental import pallas as pl

HIDDEN = 2816
NUM_EXPERTS = 128
TOP_K = 8
EPS = 1e-6
BLK = 512


def _router_body(x_ref, w_ref, scale_ref, ow_ref, oi_ref):
    x = x_ref[...]
    rms = jax.lax.rsqrt(jnp.mean(x * x, axis=1, keepdims=True) + EPS)
    normed = ((x * rms) * jnp.float32(HIDDEN ** -0.5)) * scale_ref[...]
    scores = jax.lax.dot_general(
        normed, w_ref[...],
        dimension_numbers=(((1,), (1,)), ((), ())),
        preferred_element_type=jnp.float32,
    )

    m = jnp.max(scores, axis=1, keepdims=True)
    unnorm = jnp.exp(scores - m)
    probs = unnorm / jnp.sum(unnorm, axis=1, keepdims=True)

    lane = jax.lax.broadcasted_iota(jnp.int32, probs.shape, 1)
    vals = probs
    tops = []
    idxs = []
    for _ in range(TOP_K):
        mk = jnp.max(vals, axis=1, keepdims=True)
        idx = jnp.min(jnp.where(vals == mk, lane, NUM_EXPERTS), axis=1,
                      keepdims=True)
        tops.append(mk)
        idxs.append(idx)
        vals = jnp.where(lane == idx, jnp.float32(-jnp.inf), vals)
    top = jnp.concatenate(tops, axis=1)
    ow_ref[...] = top / jnp.sum(top, axis=1, keepdims=True)
    oi_ref[...] = jnp.concatenate(idxs, axis=1)


@jax.jit
def kernel(hidden_states, W, scale):
    b, s, h = hidden_states.shape
    n_tok = b * s
    x = hidden_states.reshape(n_tok, h)
    grid = (n_tok // BLK,)
    ow, oi = pl.pallas_call(
        _router_body,
        grid=grid,
        in_specs=[
            pl.BlockSpec((BLK, h), lambda i: (i, 0)),
            pl.BlockSpec((NUM_EXPERTS, h), lambda i: (0, 0)),
            pl.BlockSpec((1, h), lambda i: (0, 0)),
        ],
        out_specs=[
            pl.BlockSpec((BLK, TOP_K), lambda i: (i, 0)),
            pl.BlockSpec((BLK, TOP_K), lambda i: (i, 0)),
        ],
        out_shape=[
            jax.ShapeDtypeStruct((n_tok, TOP_K), jnp.float32),
            jax.ShapeDtypeStruct((n_tok, TOP_K), jnp.int32),
        ],
    )(x, W, scale.reshape(1, h))
    return ow.reshape(b, s, TOP_K), oi.reshape(b, s, TOP_K)


# speedup vs baseline: 1.2207x; 1.2207x over previous
import functools

import jax
import jax.numpy as jnp
from jax.experimental import pallas as pl

HIDDEN = 2816
NUM_EXPERTS = 128
TOP_K = 8
EPS = 1e-6
BLK = 512


def _router_body(x_ref, w_ref, scale_ref, ow_ref, oi_ref):
    x = x_ref[...]
    rms = jax.lax.rsqrt(jnp.mean(x * x, axis=1, keepdims=True) + EPS)
    normed = ((x * rms) * jnp.float32(HIDDEN ** -0.5)) * scale_ref[...]
    scores = jax.lax.dot_general(
        normed, w_ref[...],
        dimension_numbers=(((1,), (1,)), ((), ())),
        preferred_element_type=jnp.float32,
    )

    m = jnp.max(scores, axis=1, keepdims=True)
    unnorm = jnp.exp(scores - m)
    probs = unnorm / jnp.sum(unnorm, axis=1, keepdims=True)

    lanef = jax.lax.broadcasted_iota(jnp.int32, probs.shape, 1).astype(
        jnp.float32)
    vals = probs
    tops = []
    idxs = []
    for _ in range(TOP_K):
        mk = jnp.max(vals, axis=1, keepdims=True)
        idx = jnp.min(jnp.where(vals == mk, lanef, jnp.float32(NUM_EXPERTS)),
                      axis=1, keepdims=True)
        tops.append(mk)
        idxs.append(idx)
        vals = jnp.where(lanef == idx, jnp.float32(-jnp.inf), vals)
    top = jnp.concatenate(tops, axis=1)
    ow_ref[...] = top / jnp.sum(top, axis=1, keepdims=True)
    oi_ref[...] = jnp.concatenate(idxs, axis=1).astype(jnp.int32)


@jax.jit
def kernel(hidden_states, W, scale):
    b, s, h = hidden_states.shape
    n_tok = b * s
    x = hidden_states.reshape(n_tok, h)
    grid = (n_tok // BLK,)
    ow, oi = pl.pallas_call(
        _router_body,
        grid=grid,
        in_specs=[
            pl.BlockSpec((BLK, h), lambda i: (i, 0)),
            pl.BlockSpec((NUM_EXPERTS, h), lambda i: (0, 0)),
            pl.BlockSpec((1, h), lambda i: (0, 0)),
        ],
        out_specs=[
            pl.BlockSpec((BLK, TOP_K), lambda i: (i, 0)),
            pl.BlockSpec((BLK, TOP_K), lambda i: (i, 0)),
        ],
        out_shape=[
            jax.ShapeDtypeStruct((n_tok, TOP_K), jnp.float32),
            jax.ShapeDtypeStruct((n_tok, TOP_K), jnp.int32),
        ],
    )(x, W, scale.reshape(1, h))
    return ow.reshape(b, s, TOP_K), oi.reshape(b, s, TOP_K)
